# baseline (speedup 1.0000x reference)
import functools

import jax
import jax.numpy as jnp
from jax.experimental import pallas as pl
from jax.experimental.pallas import tpu as pltpu


def _cnn_kernel(xc_ref, w1_ref, b1_ref, w2_ref, b2_ref, w3_ref, b3_ref,
                wf1_ref, bf1_ref, wf2_ref, bf2_ref,
                o_ref,
                pad2_ref, pad3_ref, *, b_blk):
    f32 = jnp.float32
    bf16 = jnp.bfloat16
    b = b_blk

    xc = xc_ref[...]
    a1 = (jnp.dot(xc, w1_ref[...], preferred_element_type=f32)
          + b1_ref[...]).astype(bf16)
    a1 = jnp.maximum(a1, 0).reshape(b, 16, 2, 4, 512)
    a1 = jnp.max(a1, axis=2)
    p0 = jnp.maximum(a1[..., 0:64], a1[..., 64:128])
    p1 = jnp.maximum(a1[..., 128:192], a1[..., 192:256])
    p2 = jnp.maximum(a1[..., 256:320], a1[..., 320:384])
    p3 = jnp.maximum(a1[..., 384:448], a1[..., 448:512])

    pad2_ref[:, 0:1] = jnp.zeros((b, 1, 5, 256), bf16)
    pad2_ref[:, 17:18] = jnp.zeros((b, 1, 5, 256), bf16)
    pad2_ref[:, 1:17, 0:1, 0:64] = jnp.zeros((b, 16, 1, 64), bf16)
    pad2_ref[:, 1:17, 4:5, 64:256] = jnp.zeros((b, 16, 1, 192), bf16)
    pad2_ref[:, 1:17, 0:4, 64:128] = p0
    pad2_ref[:, 1:17, 0:4, 128:192] = p1
    pad2_ref[:, 1:17, 0:4, 192:256] = p2
    pad2_ref[:, 1:17, 1:5, 0:64] = p3

    x2 = jnp.concatenate(
        [c for dy in range(3) for c in
         (pad2_ref[:, dy:dy + 16, 0:4, :].reshape(b * 16 * 4, 256),
          pad2_ref[:, dy:dy + 16, 1:5, 0:128].reshape(b * 16 * 4, 128))],
        axis=-1)
    a2 = (jnp.dot(x2, w2_ref[...], preferred_element_type=f32)
          + b2_ref[...]).astype(bf16)
    a2 = jnp.maximum(a2, 0).reshape(b, 8, 2, 4, 512)
    a2 = jnp.max(a2, axis=2)
    p2e = jnp.maximum(a2[..., 0:128], a2[..., 128:256])
    p2o = jnp.maximum(a2[..., 256:384], a2[..., 384:512])

    pad3_ref[:, 0:1] = jnp.zeros((b, 1, 5, 256), bf16)
    pad3_ref[:, 9:10] = jnp.zeros((b, 1, 5, 256), bf16)
    pad3_ref[:, 1:9, 0:1, 0:128] = jnp.zeros((b, 8, 1, 128), bf16)
    pad3_ref[:, 1:9, 4:5, 128:256] = jnp.zeros((b, 8, 1, 128), bf16)
    pad3_ref[:, 1:9, 0:4, 128:256] = p2e
    pad3_ref[:, 1:9, 1:5, 0:128] = p2o

    x3 = jnp.concatenate(
        [c for dy in range(3) for c in
         (pad3_ref[:, dy:dy + 8, 0:4, :].reshape(b * 8 * 4, 256),
          pad3_ref[:, dy:dy + 8, 1:5, :].reshape(b * 8 * 4, 256))],
        axis=-1)
    a3 = (jnp.dot(x3, w3_ref[...], preferred_element_type=f32)
          + b3_ref[...]).astype(bf16)
    a3 = jnp.maximum(a3, 0).reshape(b, 4, 2, 4, 512)
    a3 = jnp.max(a3, axis=2)
    p3f = jnp.maximum(a3[..., 0:256], a3[..., 256:512])

    feat = jnp.concatenate(
        [p3f[:, hh, ww, :] for hh in range(4) for ww in range(4)], axis=-1)
    h1 = jnp.dot(feat, wf1_ref[...], preferred_element_type=f32) + bf1_ref[...]
    h1 = jnp.maximum(h1, 0.0).astype(bf16)

    logits = jnp.dot(h1, wf2_ref[...], preferred_element_type=f32) + bf2_ref[...]
    m = jnp.max(logits, axis=-1, keepdims=True)
    lse = m + jnp.log(jnp.sum(jnp.exp(logits - m), axis=-1, keepdims=True))
    o_ref[...] = logits - lse


def _prep_weights(conv1_w, conv1_b, conv2_w, conv2_b, conv3_w, conv3_b,
                  fc1_w, fc1_b, fc2_w, fc2_b):
    bf16, f32 = jnp.bfloat16, jnp.float32

    t1 = jnp.transpose(conv1_w, (2, 3, 1, 0))
    w1 = jnp.concatenate(
        [jnp.pad(t1, ((0, 0), (j, 7 - j), (0, 0), (0, 0))).reshape(90, 64)
         for j in range(8)], axis=1).astype(bf16)
    b1 = jnp.tile(conv1_b, 8).reshape(1, 512).astype(f32)

    t2 = jnp.transpose(conv2_w, (2, 3, 1, 0))
    w2 = jnp.concatenate(
        [jnp.pad(t2, ((0, 0), (j, 3 - j), (0, 0), (0, 0))).reshape(1152, 128)
         for j in range(4)], axis=1).astype(bf16)
    b2 = jnp.tile(conv2_b, 4).reshape(1, 512).astype(f32)

    t3 = jnp.transpose(conv3_w, (2, 3, 1, 0))
    w3 = jnp.concatenate(
        [jnp.pad(t3, ((0, 0), (j, 1 - j), (0, 0), (0, 0))).reshape(1536, 256)
         for j in range(2)], axis=1).astype(bf16)
    b3 = jnp.tile(conv3_b, 2).reshape(1, 512).astype(f32)

    wf1 = fc1_w.T.reshape(256, 4, 4, 512)
    wf1 = jnp.transpose(wf1, (1, 2, 0, 3)).reshape(4096, 512).astype(bf16)
    bf1 = fc1_b.reshape(1, 512).astype(f32)

    wf2 = jnp.pad(fc2_w.T, ((0, 0), (0, 118))).astype(bf16)
    bf2 = jnp.pad(fc2_b, (0, 118), constant_values=-1e9)
    bf2 = bf2.reshape(1, 128).astype(f32)

    return (w1, b1, w2, b2, w3, b3, wf1, bf1, wf2, bf2)


def kernel(conv1_w, conv1_b, conv2_w, conv2_b, conv3_w, conv3_b,
           fc1_w, fc1_b, fc2_w, fc2_b, x_nchw, *, block_b=32):
    w = _prep_weights(conv1_w, conv1_b, conv2_w, conv2_b, conv3_w, conv3_b,
                      fc1_w, fc1_b, fc2_w, fc2_b)
    B = x_nchw.shape[0]

    x = jnp.transpose(x_nchw, (0, 2, 3, 1)).astype(jnp.float32)
    xf = x.reshape(B, 32, 96)
    xf = jnp.pad(xf, ((0, 0), (1, 1), (3, 3))).astype(jnp.bfloat16)
    x_col = jnp.stack(
        [jnp.concatenate(
            [xf[:, dy:dy + 32, 24 * g:24 * g + 30] for dy in range(3)],
            axis=-1)
         for g in range(4)], axis=2)
    x_col = x_col.reshape(B * 128, 90)

    b_blk = max(1, min(int(block_b), -(-B // 2)))
    pad_b = (-B) % (2 * b_blk)
    if pad_b:
        x_col = jnp.pad(x_col, ((0, pad_b * 128), (0, 0)))
    n_tiles = (B + pad_b) // b_blk
    half = n_tiles // 2

    const = dict(pipeline_mode=pl.Buffered(1))
    body = functools.partial(_cnn_kernel, b_blk=b_blk)
    out = pl.pallas_call(
        body,
        out_shape=jax.ShapeDtypeStruct((B + pad_b, 128), jnp.float32),
        grid=(2, half),
        in_specs=[
            pl.BlockSpec((b_blk * 128, 90),
                         lambda i, j, h=half: (i * h + j, 0)),
            pl.BlockSpec((90, 512), lambda i, j: (0, 0), **const),
            pl.BlockSpec((1, 512), lambda i, j: (0, 0), **const),
            pl.BlockSpec((1152, 512), lambda i, j: (0, 0), **const),
            pl.BlockSpec((1, 512), lambda i, j: (0, 0), **const),
            pl.BlockSpec((1536, 512), lambda i, j: (0, 0), **const),
            pl.BlockSpec((1, 512), lambda i, j: (0, 0), **const),
            pl.BlockSpec((4096, 512), lambda i, j: (0, 0), **const),
            pl.BlockSpec((1, 512), lambda i, j: (0, 0), **const),
            pl.BlockSpec((512, 128), lambda i, j: (0, 0), **const),
            pl.BlockSpec((1, 128), lambda i, j: (0, 0), **const),
        ],
        out_specs=pl.BlockSpec((b_blk, 128), lambda i, j, h=half: (i * h + j, 0)),
        scratch_shapes=[
            pltpu.VMEM((b_blk, 18, 5, 256), jnp.bfloat16),
            pltpu.VMEM((b_blk, 10, 5, 256), jnp.bfloat16),
        ],
        compiler_params=pltpu.CompilerParams(
            dimension_semantics=("parallel", "arbitrary"),
            vmem_limit_bytes=48 * 1024 * 1024),
    )(x_col, *w)
    return out[:B, :10]


# speedup vs baseline: 4.7679x; 4.7679x over previous
import functools

import jax
import jax.numpy as jnp
from jax.experimental import pallas as pl
from jax.experimental.pallas import tpu as pltpu


def _cnn_kernel(xc_ref, w1_ref, b1_ref, w2_ref, b2_ref, w3_ref, b3_ref,
                wf1_ref, bf1_ref, wf2_ref, bf2_ref,
                o_ref,
                pad2_ref, pad3_ref, *, b_blk):
    f32 = jnp.float32
    bf16 = jnp.bfloat16
    b = b_blk

    xc = xc_ref[...].reshape(b * 32 * 4, 128)
    a1 = (jnp.dot(xc, w1_ref[...], preferred_element_type=f32)
          + b1_ref[...]).astype(bf16)
    a1 = jnp.maximum(a1, 0).reshape(b, 16, 2, 4, 512)
    a1 = jnp.max(a1, axis=2)
    p0 = jnp.maximum(a1[..., 0:64], a1[..., 64:128])
    p1 = jnp.maximum(a1[..., 128:192], a1[..., 192:256])
    p2 = jnp.maximum(a1[..., 256:320], a1[..., 320:384])
    p3 = jnp.maximum(a1[..., 384:448], a1[..., 448:512])

    pad2_ref[:, 0:1] = jnp.zeros((b, 1, 5, 256), bf16)
    pad2_ref[:, 17:18] = jnp.zeros((b, 1, 5, 256), bf16)
    pad2_ref[:, 1:17, 0:1, 0:64] = jnp.zeros((b, 16, 1, 64), bf16)
    pad2_ref[:, 1:17, 4:5, 64:256] = jnp.zeros((b, 16, 1, 192), bf16)
    pad2_ref[:, 1:17, 0:4, 64:128] = p0
    pad2_ref[:, 1:17, 0:4, 128:192] = p1
    pad2_ref[:, 1:17, 0:4, 192:256] = p2
    pad2_ref[:, 1:17, 1:5, 0:64] = p3

    x2 = jnp.concatenate(
        [c for dy in range(3) for c in
         (pad2_ref[:, dy:dy + 16, 0:4, :].reshape(b * 16 * 4, 256),
          pad2_ref[:, dy:dy + 16, 1:5, 0:128].reshape(b * 16 * 4, 128))],
        axis=-1)
    a2 = (jnp.dot(x2, w2_ref[...], preferred_element_type=f32)
          + b2_ref[...]).astype(bf16)
    a2 = jnp.maximum(a2, 0).reshape(b, 8, 2, 4, 512)
    a2 = jnp.max(a2, axis=2)
    p2e = jnp.maximum(a2[..., 0:128], a2[..., 128:256])
    p2o = jnp.maximum(a2[..., 256:384], a2[..., 384:512])

    pad3_ref[:, 0:1] = jnp.zeros((b, 1, 5, 256), bf16)
    pad3_ref[:, 9:10] = jnp.zeros((b, 1, 5, 256), bf16)
    pad3_ref[:, 1:9, 0:1, 0:128] = jnp.zeros((b, 8, 1, 128), bf16)
    pad3_ref[:, 1:9, 4:5, 128:256] = jnp.zeros((b, 8, 1, 128), bf16)
    pad3_ref[:, 1:9, 0:4, 128:256] = p2e
    pad3_ref[:, 1:9, 1:5, 0:128] = p2o

    x3 = jnp.concatenate(
        [c for dy in range(3) for c in
         (pad3_ref[:, dy:dy + 8, 0:4, :].reshape(b * 8 * 4, 256),
          pad3_ref[:, dy:dy + 8, 1:5, :].reshape(b * 8 * 4, 256))],
        axis=-1)
    a3 = (jnp.dot(x3, w3_ref[...], preferred_element_type=f32)
          + b3_ref[...]).astype(bf16)
    a3 = jnp.maximum(a3, 0).reshape(b, 4, 2, 4, 512)
    a3 = jnp.max(a3, axis=2)
    p3f = jnp.maximum(a3[..., 0:256], a3[..., 256:512])

    feat = jnp.concatenate(
        [p3f[:, hh, ww, :] for hh in range(4) for ww in range(4)], axis=-1)
    h1 = jnp.dot(feat, wf1_ref[...], preferred_element_type=f32) + bf1_ref[...]
    h1 = jnp.maximum(h1, 0.0).astype(bf16)

    logits = jnp.dot(h1, wf2_ref[...], preferred_element_type=f32) + bf2_ref[...]
    m = jnp.max(logits, axis=-1, keepdims=True)
    lse = m + jnp.log(jnp.sum(jnp.exp(logits - m), axis=-1, keepdims=True))
    o_ref[...] = logits - lse


def _prep_weights(conv1_w, conv1_b, conv2_w, conv2_b, conv3_w, conv3_b,
                  fc1_w, fc1_b, fc2_w, fc2_b):
    bf16, f32 = jnp.bfloat16, jnp.float32

    t1 = jnp.transpose(conv1_w, (2, 3, 1, 0))
    w1 = jnp.concatenate(
        [jnp.pad(t1, ((0, 0), (j, 7 - j), (0, 0), (0, 0))).reshape(90, 64)
         for j in range(8)], axis=1)
    w1 = jnp.pad(w1, ((0, 38), (0, 0))).astype(bf16)
    b1 = jnp.tile(conv1_b, 8).reshape(1, 512).astype(f32)

    t2 = jnp.transpose(conv2_w, (2, 3, 1, 0))
    w2 = jnp.concatenate(
        [jnp.pad(t2, ((0, 0), (j, 3 - j), (0, 0), (0, 0))).reshape(1152, 128)
         for j in range(4)], axis=1).astype(bf16)
    b2 = jnp.tile(conv2_b, 4).reshape(1, 512).astype(f32)

    t3 = jnp.transpose(conv3_w, (2, 3, 1, 0))
    w3 = jnp.concatenate(
        [jnp.pad(t3, ((0, 0), (j, 1 - j), (0, 0), (0, 0))).reshape(1536, 256)
         for j in range(2)], axis=1).astype(bf16)
    b3 = jnp.tile(conv3_b, 2).reshape(1, 512).astype(f32)

    wf1 = fc1_w.T.reshape(256, 4, 4, 512)
    wf1 = jnp.transpose(wf1, (1, 2, 0, 3)).reshape(4096, 512).astype(bf16)
    bf1 = fc1_b.reshape(1, 512).astype(f32)

    wf2 = jnp.pad(fc2_w.T, ((0, 0), (0, 118))).astype(bf16)
    bf2 = jnp.pad(fc2_b, (0, 118), constant_values=-1e9)
    bf2 = bf2.reshape(1, 128).astype(f32)

    return (w1, b1, w2, b2, w3, b3, wf1, bf1, wf2, bf2)


def kernel(conv1_w, conv1_b, conv2_w, conv2_b, conv3_w, conv3_b,
           fc1_w, fc1_b, fc2_w, fc2_b, x_nchw, *, block_b=32):
    w = _prep_weights(conv1_w, conv1_b, conv2_w, conv2_b, conv3_w, conv3_b,
                      fc1_w, fc1_b, fc2_w, fc2_b)
    B = x_nchw.shape[0]

    x = jnp.transpose(x_nchw, (0, 2, 3, 1)).astype(jnp.float32)
    xf = x.reshape(B, 32, 96)
    xf = jnp.pad(xf, ((0, 0), (1, 1), (3, 3))).astype(jnp.bfloat16)
    x_col = jnp.stack(
        [jnp.concatenate(
            [xf[:, dy:dy + 32, 24 * g:24 * g + 30] for dy in range(3)],
            axis=-1)
         for g in range(4)], axis=2)
    x_col = jnp.pad(x_col, ((0, 0), (0, 0), (0, 0), (0, 38)))

    b_blk = max(1, min(int(block_b), -(-B // 2)))
    pad_b = (-B) % (2 * b_blk)
    if pad_b:
        x_col = jnp.pad(x_col, ((0, pad_b), (0, 0), (0, 0), (0, 0)))
    n_tiles = (B + pad_b) // b_blk
    half = n_tiles // 2

    const = dict(pipeline_mode=pl.Buffered(1))
    body = functools.partial(_cnn_kernel, b_blk=b_blk)
    out = pl.pallas_call(
        body,
        out_shape=jax.ShapeDtypeStruct((B + pad_b, 128), jnp.float32),
        grid=(2, half),
        in_specs=[
            pl.BlockSpec((b_blk, 32, 4, 128),
                         lambda i, j, h=half: (i * h + j, 0, 0, 0)),
            pl.BlockSpec((128, 512), lambda i, j: (0, 0), **const),
            pl.BlockSpec((1, 512), lambda i, j: (0, 0), **const),
            pl.BlockSpec((1152, 512), lambda i, j: (0, 0), **const),
            pl.BlockSpec((1, 512), lambda i, j: (0, 0), **const),
            pl.BlockSpec((1536, 512), lambda i, j: (0, 0), **const),
            pl.BlockSpec((1, 512), lambda i, j: (0, 0), **const),
            pl.BlockSpec((4096, 512), lambda i, j: (0, 0), **const),
            pl.BlockSpec((1, 512), lambda i, j: (0, 0), **const),
            pl.BlockSpec((512, 128), lambda i, j: (0, 0), **const),
            pl.BlockSpec((1, 128), lambda i, j: (0, 0), **const),
        ],
        out_specs=pl.BlockSpec((b_blk, 128), lambda i, j, h=half: (i * h + j, 0)),
        scratch_shapes=[
            pltpu.VMEM((b_blk, 18, 5, 256), jnp.bfloat16),
            pltpu.VMEM((b_blk, 10, 5, 256), jnp.bfloat16),
        ],
        compiler_params=pltpu.CompilerParams(
            dimension_semantics=("parallel", "arbitrary"),
            vmem_limit_bytes=48 * 1024 * 1024),
    )(x_col, *w)
    return out[:B, :10]


# speedup vs baseline: 4.8003x; 1.0068x over previous
import functools

import jax
import jax.numpy as jnp
from jax.experimental import pallas as pl
from jax.experimental.pallas import tpu as pltpu


def _cnn_kernel(xc_ref, w1_ref, b1_ref, w2_ref, b2_ref, w3_ref, b3_ref,
                wf1_ref, bf1_ref, wf2_ref, bf2_ref,
                o_ref,
                pad2_ref, pad3_ref, *, b_blk):
    f32 = jnp.float32
    bf16 = jnp.bfloat16
    b = b_blk

    xc = xc_ref[...].reshape(b * 32 * 4, 128)
    a1 = (jnp.dot(xc, w1_ref[...], preferred_element_type=f32)
          + b1_ref[...]).astype(bf16)
    a1 = jnp.maximum(a1, 0).reshape(b, 16, 2, 4, 512)
    a1 = jnp.max(a1, axis=2)
    p0 = jnp.maximum(a1[..., 0:64], a1[..., 64:128])
    p1 = jnp.maximum(a1[..., 128:192], a1[..., 192:256])
    p2 = jnp.maximum(a1[..., 256:320], a1[..., 320:384])
    p3 = jnp.maximum(a1[..., 384:448], a1[..., 448:512])

    pad2_ref[:, 0:1] = jnp.zeros((b, 1, 5, 256), bf16)
    pad2_ref[:, 17:18] = jnp.zeros((b, 1, 5, 256), bf16)
    pad2_ref[:, 1:17, 0:1, 0:64] = jnp.zeros((b, 16, 1, 64), bf16)
    pad2_ref[:, 1:17, 4:5, 64:256] = jnp.zeros((b, 16, 1, 192), bf16)
    pad2_ref[:, 1:17, 0:4, 64:128] = p0
    pad2_ref[:, 1:17, 0:4, 128:192] = p1
    pad2_ref[:, 1:17, 0:4, 192:256] = p2
    pad2_ref[:, 1:17, 1:5, 0:64] = p3

    x2 = jnp.concatenate(
        [c for dy in range(3) for c in
         (pad2_ref[:, dy:dy + 16, 0:4, :].reshape(b * 16 * 4, 256),
          pad2_ref[:, dy:dy + 16, 1:5, 0:128].reshape(b * 16 * 4, 128))],
        axis=-1)
    a2 = (jnp.dot(x2, w2_ref[...], preferred_element_type=f32)
          + b2_ref[...]).astype(bf16)
    a2 = jnp.maximum(a2, 0).reshape(b, 8, 2, 4, 512)
    a2 = jnp.max(a2, axis=2)
    p2e = jnp.maximum(a2[..., 0:128], a2[..., 128:256])
    p2o = jnp.maximum(a2[..., 256:384], a2[..., 384:512])

    pad3_ref[:, 0:1] = jnp.zeros((b, 1, 5, 256), bf16)
    pad3_ref[:, 9:10] = jnp.zeros((b, 1, 5, 256), bf16)
    pad3_ref[:, 1:9, 0:1, 0:128] = jnp.zeros((b, 8, 1, 128), bf16)
    pad3_ref[:, 1:9, 4:5, 128:256] = jnp.zeros((b, 8, 1, 128), bf16)
    pad3_ref[:, 1:9, 0:4, 128:256] = p2e
    pad3_ref[:, 1:9, 1:5, 0:128] = p2o

    x3 = jnp.concatenate(
        [c for dy in range(3) for c in
         (pad3_ref[:, dy:dy + 8, 0:4, :].reshape(b * 8 * 4, 256),
          pad3_ref[:, dy:dy + 8, 1:5, :].reshape(b * 8 * 4, 256))],
        axis=-1)
    a3 = (jnp.dot(x3, w3_ref[...], preferred_element_type=f32)
          + b3_ref[...]).astype(bf16)
    a3 = jnp.maximum(a3, 0).reshape(b, 4, 2, 4, 512)
    a3 = jnp.max(a3, axis=2)
    p3f = jnp.maximum(a3[..., 0:256], a3[..., 256:512])

    feat = jnp.concatenate(
        [p3f[:, hh, ww, :] for hh in range(4) for ww in range(4)], axis=-1)
    h1 = jnp.dot(feat, wf1_ref[...], preferred_element_type=f32) + bf1_ref[...]
    h1 = jnp.maximum(h1, 0.0).astype(bf16)

    logits = jnp.dot(h1, wf2_ref[...], preferred_element_type=f32) + bf2_ref[...]
    m = jnp.max(logits, axis=-1, keepdims=True)
    lse = m + jnp.log(jnp.sum(jnp.exp(logits - m), axis=-1, keepdims=True))
    o_ref[...] = logits - lse


def _prep_weights(conv1_w, conv1_b, conv2_w, conv2_b, conv3_w, conv3_b,
                  fc1_w, fc1_b, fc2_w, fc2_b):
    bf16, f32 = jnp.bfloat16, jnp.float32

    t1 = jnp.transpose(conv1_w, (2, 3, 1, 0))
    w1 = jnp.concatenate(
        [jnp.pad(t1, ((0, 0), (j, 7 - j), (0, 0), (0, 0))).reshape(90, 64)
         for j in range(8)], axis=1)
    w1 = jnp.pad(w1, ((0, 38), (0, 0))).astype(bf16)
    b1 = jnp.tile(conv1_b, 8).reshape(1, 512).astype(f32)

    t2 = jnp.transpose(conv2_w, (2, 3, 1, 0))
    w2 = jnp.concatenate(
        [jnp.pad(t2, ((0, 0), (j, 3 - j), (0, 0), (0, 0))).reshape(1152, 128)
         for j in range(4)], axis=1).astype(bf16)
    b2 = jnp.tile(conv2_b, 4).reshape(1, 512).astype(f32)

    t3 = jnp.transpose(conv3_w, (2, 3, 1, 0))
    w3 = jnp.concatenate(
        [jnp.pad(t3, ((0, 0), (j, 1 - j), (0, 0), (0, 0))).reshape(1536, 256)
         for j in range(2)], axis=1).astype(bf16)
    b3 = jnp.tile(conv3_b, 2).reshape(1, 512).astype(f32)

    wf1 = fc1_w.T.reshape(256, 4, 4, 512)
    wf1 = jnp.transpose(wf1, (1, 2, 0, 3)).reshape(4096, 512).astype(bf16)
    bf1 = fc1_b.reshape(1, 512).astype(f32)

    wf2 = jnp.pad(fc2_w.T, ((0, 0), (0, 118))).astype(bf16)
    bf2 = jnp.pad(fc2_b, (0, 118), constant_values=-1e9)
    bf2 = bf2.reshape(1, 128).astype(f32)

    return (w1, b1, w2, b2, w3, b3, wf1, bf1, wf2, bf2)


def kernel(conv1_w, conv1_b, conv2_w, conv2_b, conv3_w, conv3_b,
           fc1_w, fc1_b, fc2_w, fc2_b, x_nchw, *, block_b=32):
    w = _prep_weights(conv1_w, conv1_b, conv2_w, conv2_b, conv3_w, conv3_b,
                      fc1_w, fc1_b, fc2_w, fc2_b)
    B = x_nchw.shape[0]

    x = jnp.transpose(x_nchw, (0, 2, 3, 1)).astype(jnp.float32)
    xf = x.reshape(B, 32, 96)
    xf = jnp.pad(xf, ((0, 0), (1, 1), (3, 3))).astype(jnp.bfloat16)
    x_col = jnp.stack(
        [jnp.concatenate(
            [xf[:, dy:dy + 32, 24 * g:24 * g + 30] for dy in range(3)],
            axis=-1)
         for g in range(4)], axis=2)
    x_col = jnp.pad(x_col, ((0, 0), (0, 0), (0, 0), (0, 38)))

    b_blk = max(1, min(int(block_b), -(-B // 2)))
    pad_b = (-B) % (2 * b_blk)
    if pad_b:
        x_col = jnp.pad(x_col, ((0, pad_b), (0, 0), (0, 0), (0, 0)))
    n_tiles = (B + pad_b) // b_blk
    half = n_tiles // 2

    const = dict(pipeline_mode=pl.Buffered(1))
    body = functools.partial(_cnn_kernel, b_blk=b_blk)
    out = pl.pallas_call(
        body,
        out_shape=jax.ShapeDtypeStruct((B + pad_b, 128), jnp.float32),
        grid=(2, half),
        in_specs=[
            pl.BlockSpec((b_blk, 32, 4, 128),
                         lambda i, j, h=half: (i * h + j, 0, 0, 0)),
            pl.BlockSpec((128, 512), lambda i, j: (0, 0), **const),
            pl.BlockSpec((1, 512), lambda i, j: (0, 0), **const),
            pl.BlockSpec((1152, 512), lambda i, j: (0, 0), **const),
            pl.BlockSpec((1, 512), lambda i, j: (0, 0), **const),
            pl.BlockSpec((1536, 512), lambda i, j: (0, 0), **const),
            pl.BlockSpec((1, 512), lambda i, j: (0, 0), **const),
            pl.BlockSpec((4096, 512), lambda i, j: (0, 0), **const),
            pl.BlockSpec((1, 512), lambda i, j: (0, 0), **const),
            pl.BlockSpec((512, 128), lambda i, j: (0, 0), **const),
            pl.BlockSpec((1, 128), lambda i, j: (0, 0), **const),
        ],
        out_specs=pl.BlockSpec((b_blk, 128), lambda i, j, h=half: (i * h + j, 0)),
        scratch_shapes=[
            pltpu.VMEM((b_blk, 18, 5, 256), jnp.bfloat16),
            pltpu.VMEM((b_blk, 10, 5, 256), jnp.bfloat16),
        ],
        compiler_params=pltpu.CompilerParams(
            dimension_semantics=("parallel", "arbitrary"),
            vmem_limit_bytes=58 * 1024 * 1024),
    )(x_col, *w)
    return out[:B, :10]
